# Initial kernel scaffold; baseline (speedup 1.0000x reference)
import jax
import jax.numpy as jnp
from jax.experimental import pallas as pl


def kernel(edge_index, x, W1, b1, W2, b2, temp):
    raise NotImplementedError("write your pallas kernel here")


# speedup vs baseline: 5.3959x; 5.3959x over previous
import functools
import math

import jax
import jax.numpy as jnp
import numpy as np
from jax import lax
from jax.experimental import pallas as pl
from jax.experimental.pallas import tpu as pltpu
from jax.experimental.pallas import tpu_sc as plsc

_K = 10
_N = 10000
_FIN = 128
_HID = 64
_NCLS = 32
_E = 320000

_NC, _NS = 2, 16
_NW = _NC * _NS
_CHUNK = 128
_NCH = -(-_E // (_NW * _CHUNK))
_EPAD = _NCH * _NW * _CHUNK
_AR = 10240
_ZR = _AR // _NS
_HW = 128
_BLK = 1000
_GRID = _N // _BLK


def _cheby_t(i, x):
    if i == 0:
        return 1.0
    t0, t1 = 1.0, x
    for _ in range(2, i + 1):
        t0, t1 = t1, 2.0 * x * t1 - t0
    return t1


def _interp_matrix(k):
    xs = [math.cos((k - j + 0.5) * math.pi / (k + 1)) for j in range(k + 1)]
    return np.array([[_cheby_t(i, xs[j]) for j in range(k + 1)]
                     for i in range(k + 1)], dtype=np.float32)

_M_INTERP = _interp_matrix(_K)

_MESH = plsc.VectorSubcoreMesh(core_axis_name="c", subcore_axis_name="s",
                               num_cores=_NC, num_subcores=_NS)


@functools.partial(
    pl.kernel,
    out_type=jax.ShapeDtypeStruct((_NC * _AR, _HW), jnp.float32),
    mesh=_MESH,
    scratch_types=[
        pltpu.VMEM((_NCH, _CHUNK), jnp.int32),
        pltpu.VMEM((_NCH, _CHUNK), jnp.int32),
        pltpu.VMEM((_CHUNK, _HW), jnp.float32),
        pltpu.VMEM_SHARED((_AR, _HW), jnp.float32),
        pltpu.SemaphoreType.DMA,
    ],
)
def _prop_sc(zs, srcg, dsts, zrows, out, sidx, didx, rows, acc, sem):
    c = lax.axis_index("c")
    s = lax.axis_index("s")
    w = c * _NS + s
    pltpu.sync_copy(srcg.at[w], sidx)
    pltpu.sync_copy(dsts.at[w], didx)
    pltpu.sync_copy(zrows, acc.at[pl.ds(s * _ZR, _ZR)])
    plsc.subcore_barrier()

    def body(j, carry):
        pltpu.async_copy(zs.at[sidx.at[j]], rows, sem).wait()
        pltpu.sync_copy(rows, acc.at[didx.at[j]], add=True)
        return carry

    lax.fori_loop(0, _NCH, body, 0)
    plsc.subcore_barrier()
    pltpu.sync_copy(acc.at[pl.ds(s * _ZR, _ZR)],
                    out.at[pl.ds(c * _AR + s * _ZR, _ZR)])


def _prologue_body(xr, w1r, b1r, degr, hr, zsr, dbr):
    h = jnp.maximum(
        jnp.dot(xr[...], w1r[...], preferred_element_type=jnp.float32)
        + b1r[...], 0.0)
    d = degr[0] + degr[1]
    dis = jnp.where(d > 0.0, lax.rsqrt(jnp.where(d > 0.0, d, 1.0)), 0.0)
    hr[...] = h
    zsr[:, :_HID] = dis[:, :_HID] * h
    zsr[:, _HID:] = jnp.zeros((_BLK, _HW - _HID), jnp.float32)
    dbr[...] = dis


_prologue_tc = pl.pallas_call(
    _prologue_body,
    grid=(_GRID,),
    in_specs=[
        pl.BlockSpec((_BLK, _FIN), lambda i: (i, 0)),
        pl.BlockSpec((_FIN, _HID), lambda i: (0, 0)),
        pl.BlockSpec((1, _HID), lambda i: (0, 0)),
        pl.BlockSpec((2, _BLK, _HW), lambda i: (0, i, 0)),
    ],
    out_specs=[
        pl.BlockSpec((_BLK, _HID), lambda i: (i, 0)),
        pl.BlockSpec((_BLK, _HW), lambda i: (i, 0)),
        pl.BlockSpec((_BLK, _HW), lambda i: (i, 0)),
    ],
    out_shape=[
        jax.ShapeDtypeStruct((_N, _HID), jnp.float32),
        jax.ShapeDtypeStruct((_N, _HW), jnp.float32),
        jax.ShapeDtypeStruct((_N, _HW), jnp.float32),
    ],
)


def _zs_store(zsr, db64, t):
    zsr[:, :_HID] = db64 * t
    zsr[:, _HID:] = jnp.zeros((_BLK, _HW - _HID), jnp.float32)


def _step1_body(accr, dbr, h0r, scr, txr, outr, zsr):
    db64 = dbr[:, :_HID]
    p = -db64 * (accr[0][:, :_HID] + accr[1][:, :_HID])
    txr[...] = p
    outr[...] = scr[0] * h0r[...] + scr[1] * p
    _zs_store(zsr, db64, p)


_step1_tc = pl.pallas_call(
    _step1_body,
    grid=(_GRID,),
    in_specs=[
        pl.BlockSpec((2, _BLK, _HW), lambda i: (0, i, 0)),
        pl.BlockSpec((_BLK, _HW), lambda i: (i, 0)),
        pl.BlockSpec((_BLK, _HID), lambda i: (i, 0)),
        pl.BlockSpec(memory_space=pltpu.SMEM),
    ],
    out_specs=[
        pl.BlockSpec((_BLK, _HID), lambda i: (i, 0)),
        pl.BlockSpec((_BLK, _HID), lambda i: (i, 0)),
        pl.BlockSpec((_BLK, _HW), lambda i: (i, 0)),
    ],
    out_shape=[
        jax.ShapeDtypeStruct((_N, _HID), jnp.float32),
        jax.ShapeDtypeStruct((_N, _HID), jnp.float32),
        jax.ShapeDtypeStruct((_N, _HW), jnp.float32),
    ],
)


def _stepmid_body(accr, dbr, tx0r, outpr, cr, txr, outr, zsr):
    db64 = dbr[:, :_HID]
    p = -db64 * (accr[0][:, :_HID] + accr[1][:, :_HID])
    t = 2.0 * p - tx0r[...]
    txr[...] = t
    outr[...] = outpr[...] + cr[0] * t
    _zs_store(zsr, db64, t)


_stepmid_tc = pl.pallas_call(
    _stepmid_body,
    grid=(_GRID,),
    in_specs=[
        pl.BlockSpec((2, _BLK, _HW), lambda i: (0, i, 0)),
        pl.BlockSpec((_BLK, _HW), lambda i: (i, 0)),
        pl.BlockSpec((_BLK, _HID), lambda i: (i, 0)),
        pl.BlockSpec((_BLK, _HID), lambda i: (i, 0)),
        pl.BlockSpec(memory_space=pltpu.SMEM),
    ],
    out_specs=[
        pl.BlockSpec((_BLK, _HID), lambda i: (i, 0)),
        pl.BlockSpec((_BLK, _HID), lambda i: (i, 0)),
        pl.BlockSpec((_BLK, _HW), lambda i: (i, 0)),
    ],
    out_shape=[
        jax.ShapeDtypeStruct((_N, _HID), jnp.float32),
        jax.ShapeDtypeStruct((_N, _HID), jnp.float32),
        jax.ShapeDtypeStruct((_N, _HW), jnp.float32),
    ],
)


def _steplast_body(accr, dbr, tx0r, outpr, cr, w2r, b2r, houtr, yr):
    db64 = dbr[:, :_HID]
    p = -db64 * (accr[0][:, :_HID] + accr[1][:, :_HID])
    t = 2.0 * p - tx0r[...]
    h = outpr[...] + cr[0] * t
    houtr[...] = h
    yr[...] = jnp.dot(h, w2r[...], preferred_element_type=jnp.float32) + b2r[...]


_steplast_tc = pl.pallas_call(
    _steplast_body,
    grid=(_GRID,),
    in_specs=[
        pl.BlockSpec((2, _BLK, _HW), lambda i: (0, i, 0)),
        pl.BlockSpec((_BLK, _HW), lambda i: (i, 0)),
        pl.BlockSpec((_BLK, _HID), lambda i: (i, 0)),
        pl.BlockSpec((_BLK, _HID), lambda i: (i, 0)),
        pl.BlockSpec(memory_space=pltpu.SMEM),
        pl.BlockSpec((_HID, _NCLS), lambda i: (0, 0)),
        pl.BlockSpec((1, _NCLS), lambda i: (0, 0)),
    ],
    out_specs=[
        pl.BlockSpec((_BLK, _HID), lambda i: (i, 0)),
        pl.BlockSpec((_BLK, _NCLS), lambda i: (i, 0)),
    ],
    out_shape=[
        jax.ShapeDtypeStruct((_N, _HID), jnp.float32),
        jax.ShapeDtypeStruct((_N, _NCLS), jnp.float32),
    ],
)


def kernel(edge_index, x, W1, b1, W2, b2, temp):
    src = edge_index[0]
    dst = edge_index[1]
    npad = _EPAD - _E
    srcg = jnp.concatenate([src, jnp.zeros((npad,), jnp.int32)]).reshape(
        _NW, _NCH, _CHUNK)
    trash = jnp.full((npad,), _N, jnp.int32)
    dsts = jnp.concatenate([dst, trash]).reshape(_NW, _NCH, _CHUNK)
    srcs = jnp.concatenate([src, trash]).reshape(_NW, _NCH, _CHUNK)

    onesm = jnp.ones((_N, _HW), jnp.float32)
    zrows = jnp.zeros((_ZR, _HW), jnp.float32)

    coe = (2.0 / (_K + 1)) * (jnp.asarray(_M_INTERP) @ jnp.maximum(temp, 0.0))

    degp = _prop_sc(onesm, srcg, srcs, zrows).reshape(_NC, _AR, _HW)
    h0, zs, db = _prologue_tc(x, W1, b1.reshape(1, _HID), degp)

    acc = _prop_sc(zs, srcg, dsts, zrows).reshape(_NC, _AR, _HW)
    sc1 = jnp.stack([coe[0] * 0.5, coe[1]])
    tx1, out, zs = _step1_tc(acc, db, h0, sc1)
    tx0 = h0
    for i in range(2, _K):
        acc = _prop_sc(zs, srcg, dsts, zrows).reshape(_NC, _AR, _HW)
        tx2, out, zs = _stepmid_tc(acc, db, tx0, out, coe[i:i + 1])
        tx0 = tx1
        tx1 = tx2
    acc = _prop_sc(zs, srcg, dsts, zrows).reshape(_NC, _AR, _HW)
    h_out, y = _steplast_tc(acc, db, tx0, out, coe[_K:_K + 1], W2,
                            b2.reshape(1, _NCLS))
    return (y, h_out)
